# baseline (speedup 1.0000x reference)
import functools

import jax
import jax.numpy as jnp
from jax import lax
from jax.experimental import pallas as pl
from jax.experimental.pallas import tpu as pltpu
from jax.experimental.pallas import tpu_sc as plsc

NC = 2
NS = 16
NW = NC * NS
LANES = 16

SEQ = 200
EMB = 32
JB = EMB // 8
SR = SEQ // 8


@functools.lru_cache(maxsize=None)
def _make(batch: int):
  nb = batch // 128
  ct = nb // NW

  mesh = plsc.VectorSubcoreMesh(core_axis_name="c", subcore_axis_name="s")

  @functools.partial(
      pl.kernel,
      out_type=jax.ShapeDtypeStruct((SEQ, JB, nb, 8, 128), jnp.float32),
      mesh=mesh,
      scratch_types=[
          pltpu.VMEM((ct, 8, 128), jnp.int32),
          pltpu.VMEM((128, EMB), jnp.float32),
          pltpu.VMEM((JB, ct, 8, 128), jnp.float32),
          pltpu.VMEM((SEQ * EMB,), jnp.float32),
          pltpu.SemaphoreType.DMA,
      ],
      compiler_params=pltpu.CompilerParams(use_tc_tiling_on_sc=False,
                                           needs_layout_passes=False),
  )
  def body(idx_hbm, table_hbm, pos_hbm, out_hbm, idx_v, rows_v, out_v, pos_v,
           sem):
    wid = lax.axis_index("s") * NC + lax.axis_index("c")
    pltpu.sync_copy(pos_hbm, pos_v)
    iota = lax.iota(jnp.int32, LANES)

    def r_body(r, carry):
      pltpu.sync_copy(idx_hbm.at[r, pl.ds(wid * ct, ct)], idx_v)

      def si_body(si, carry2):
        s = r * 8 + si
        pj = [
            plsc.load_gather(pos_v, [jnp.full((LANES,), s * EMB + j,
                                              jnp.int32)])
            for j in range(EMB)
        ]
        for c in range(ct):
          pltpu.async_copy(
              table_hbm.at[idx_v.at[c, si]], rows_v, sem).wait()

          def g_body(g, carry3):
            rowidx = iota + g * LANES
            for jb in range(JB):
              for ji in range(8):
                j = jb * 8 + ji
                vals = plsc.load_gather(
                    rows_v, [rowidx, jnp.full((LANES,), j, jnp.int32)])
                out_v[jb, c, ji, pl.ds(g * LANES, LANES)] = vals + pj[j]
            return carry3

          lax.fori_loop(0, 8, g_body, 0)
        for jb in range(JB):
          pltpu.sync_copy(out_v.at[jb],
                          out_hbm.at[s, jb, pl.ds(wid * ct, ct)])
        return carry2

      lax.fori_loop(0, 8, si_body, 0)
      return carry

    lax.fori_loop(0, SR, r_body, 0)

  return body


def kernel(inputs, token_table, pos_table):
  batch, seq = inputs.shape
  emb = token_table.shape[1]
  idx4 = (inputs.astype(jnp.int32).T
          .reshape(seq // 8, 8, batch // 128, 128)
          .transpose(0, 2, 1, 3))
  out5 = _make(batch)(idx4, token_table, pos_table.reshape(-1))
  return out5.transpose(2, 4, 0, 1, 3).reshape(batch, seq, emb)


# speedup vs baseline: 4.6187x; 4.6187x over previous
import functools

import jax
import jax.numpy as jnp
from jax import lax
from jax.experimental import pallas as pl
from jax.experimental.pallas import tpu as pltpu
from jax.experimental.pallas import tpu_sc as plsc

NC = 2
NS = 16
NW = NC * NS
LANES = 16

SEQ = 200
EMB = 32
JB = EMB // 8
SR = SEQ // 8


@functools.lru_cache(maxsize=None)
def _make(batch: int):
  nb = batch // 128
  ct = nb // NW
  cw = ct * 128

  mesh = plsc.VectorSubcoreMesh(core_axis_name="c", subcore_axis_name="s")

  @functools.partial(
      pl.kernel,
      out_type=jax.ShapeDtypeStruct((SEQ, JB, nb, 8, 128), jnp.float32),
      mesh=mesh,
      scratch_types=[
          pltpu.VMEM((2, ct, 8, 128), jnp.int32),
          pltpu.VMEM((2, cw, EMB), jnp.float32),
          pltpu.VMEM((2, JB, ct, 8, 128), jnp.float32),
          pltpu.VMEM((SEQ * EMB,), jnp.float32),
          pltpu.SemaphoreType.DMA,
          pltpu.SemaphoreType.DMA,
          pltpu.SemaphoreType.DMA,
          pltpu.SemaphoreType.DMA,
      ],
      compiler_params=pltpu.CompilerParams(use_tc_tiling_on_sc=False,
                                           needs_layout_passes=False),
  )
  def body(idx_hbm, table_hbm, pos_hbm, out_hbm, idx_v, rows_v, out_v, pos_v,
           semg0, semg1, semo0, semo1):
    wid = lax.axis_index("s") * NC + lax.axis_index("c")
    col0 = wid * ct
    pltpu.sync_copy(pos_hbm, pos_v)
    iota = lax.iota(jnp.int32, LANES)
    semg = (semg0, semg1)
    semo = (semo0, semo1)

    def fire_gathers(t, buf):
      r = t >> 3
      si = t & 7
      slot = r & 1
      return [
          pltpu.async_copy(
              table_hbm.at[idx_v.at[slot, c, si]],
              rows_v.at[buf, pl.ds(c * 128, 128)],
              semg[buf])
          for c in range(ct)
      ]

    def half(t, buf):
      si = t & 7
      pj = [
          plsc.load_gather(pos_v, [jnp.full((LANES,), t * EMB + j, jnp.int32)])
          for j in range(EMB)
      ]
      for cp in fire_gathers(t, buf):
        cp.wait()

      @pl.when(jnp.logical_and(si == 7, t < SEQ - 1))
      def _():
        r1 = (t >> 3) + 1
        pltpu.sync_copy(idx_hbm.at[r1, pl.ds(col0, ct)],
                        idx_v.at[r1 & 1])

      @pl.when(t >= 2)
      def _():
        pltpu.make_async_copy(
            out_hbm.at[0, :, pl.ds(0, ct)], out_v.at[buf], semo[buf]).wait()

      rows2d = rows_v.at[buf]

      @functools.partial(plsc.parallel_loop, 0, ct * 8)
      def _(i):
        c = i >> 3
        g = i & 7
        rowidx = iota + ((c << 7) + (g << 4))
        for jb in range(JB):
          for ji in range(8):
            j = jb * 8 + ji
            vals = plsc.load_gather(
                rows2d, [rowidx, jnp.full((LANES,), j, jnp.int32)])
            out_v[buf, jb, c, ji, pl.ds(g * LANES, LANES)] = vals + pj[j]

      pltpu.async_copy(out_v.at[buf],
                       out_hbm.at[t, :, pl.ds(col0, ct)], semo[buf])

    pltpu.sync_copy(idx_hbm.at[0, pl.ds(col0, ct)], idx_v.at[0])

    def u_body(u, carry):
      half(2 * u, 0)
      half(2 * u + 1, 1)
      return carry

    lax.fori_loop(0, SEQ // 2, u_body, 0)

    pltpu.make_async_copy(
        out_hbm.at[0, :, pl.ds(0, ct)], out_v.at[0], semo[0]).wait()
    pltpu.make_async_copy(
        out_hbm.at[0, :, pl.ds(0, ct)], out_v.at[1], semo[1]).wait()

  return body


def kernel(inputs, token_table, pos_table):
  batch, seq = inputs.shape
  emb = token_table.shape[1]
  idx4 = (inputs.astype(jnp.int32).T
          .reshape(seq // 8, 8, batch // 128, 128)
          .transpose(0, 2, 1, 3))
  out5 = _make(batch)(idx4, token_table, pos_table.reshape(-1))
  return out5.transpose(2, 4, 0, 1, 3).reshape(batch, seq, emb)


# speedup vs baseline: 4.6240x; 1.0012x over previous
import functools

import jax
import jax.numpy as jnp
from jax import lax
from jax.experimental import pallas as pl
from jax.experimental.pallas import tpu as pltpu
from jax.experimental.pallas import tpu_sc as plsc

NC = 2
NS = 16
NW = NC * NS
LANES = 16

SEQ = 200
EMB = 32
JB = EMB // 8
SR = SEQ // 8


@functools.lru_cache(maxsize=None)
def _make(batch: int):
  nb = batch // 128
  ct = nb // NW
  cw = ct * 128

  mesh = plsc.VectorSubcoreMesh(core_axis_name="c", subcore_axis_name="s")

  @functools.partial(
      pl.kernel,
      out_type=jax.ShapeDtypeStruct((SEQ, JB, nb, 8, 128), jnp.float32),
      mesh=mesh,
      scratch_types=[
          pltpu.VMEM((2, ct, 8, 128), jnp.int32),
          pltpu.VMEM((2, cw, EMB), jnp.float32),
          pltpu.VMEM((2, JB, ct, 8, 128), jnp.float32),
          pltpu.VMEM((SEQ * EMB,), jnp.float32),
          pltpu.SemaphoreType.DMA,
          pltpu.SemaphoreType.DMA,
          pltpu.SemaphoreType.DMA,
          pltpu.SemaphoreType.DMA,
      ],
      compiler_params=pltpu.CompilerParams(use_tc_tiling_on_sc=False,
                                           needs_layout_passes=False),
  )
  def body(idx_hbm, table_hbm, pos_hbm, out_hbm, idx_v, rows_v, out_v, pos_v,
           semg0, semg1, semo0, semo1):
    wid = lax.axis_index("s") * NC + lax.axis_index("c")
    col0 = wid * ct
    pltpu.sync_copy(pos_hbm, pos_v)
    iota = lax.iota(jnp.int32, LANES)
    semg = (semg0, semg1)
    semo = (semo0, semo1)

    def fire_gathers(t, buf):
      r = t >> 3
      si = t & 7
      slot = r & 1
      return [
          pltpu.async_copy(
              table_hbm.at[idx_v.at[slot, c, si]],
              rows_v.at[buf, pl.ds(c * 128, 128)],
              semg[buf])
          for c in range(ct)
      ]

    def half(t, buf):
      si = t & 7
      pj = [
          plsc.load_gather(pos_v, [jnp.full((LANES,), t * EMB + j, jnp.int32)])
          for j in range(EMB)
      ]
      r = t >> 3
      slot = r & 1
      for c in range(ct):
        pltpu.make_async_copy(
            table_hbm.at[idx_v.at[slot, c, si]],
            rows_v.at[buf, pl.ds(c * 128, 128)],
            semg[buf]).wait()

      @pl.when(jnp.logical_and(si == 7, t < SEQ - 1))
      def _():
        r1 = (t >> 3) + 1
        pltpu.sync_copy(idx_hbm.at[r1, pl.ds(col0, ct)],
                        idx_v.at[r1 & 1])

      fire_gathers(jnp.minimum(t + 1, SEQ - 1), 1 - buf)

      @pl.when(t >= 2)
      def _():
        pltpu.make_async_copy(
            out_hbm.at[0, :, pl.ds(0, ct)], out_v.at[buf], semo[buf]).wait()

      rows2d = rows_v.at[buf]

      @functools.partial(plsc.parallel_loop, 0, ct * 8)
      def _(i):
        c = i >> 3
        g = i & 7
        rowidx = iota + ((c << 7) + (g << 4))
        for jb in range(JB):
          for ji in range(8):
            j = jb * 8 + ji
            vals = plsc.load_gather(
                rows2d, [rowidx, jnp.full((LANES,), j, jnp.int32)])
            out_v[buf, jb, c, ji, pl.ds(g * LANES, LANES)] = vals + pj[j]

      pltpu.async_copy(out_v.at[buf],
                       out_hbm.at[t, :, pl.ds(col0, ct)], semo[buf])

    pltpu.sync_copy(idx_hbm.at[0, pl.ds(col0, ct)], idx_v.at[0])
    fire_gathers(0, 0)

    def u_body(u, carry):
      half(2 * u, 0)
      half(2 * u + 1, 1)
      return carry

    lax.fori_loop(0, SEQ // 2, u_body, 0)

    pltpu.make_async_copy(
        out_hbm.at[0, :, pl.ds(0, ct)], out_v.at[0], semo[0]).wait()
    pltpu.make_async_copy(
        out_hbm.at[0, :, pl.ds(0, ct)], out_v.at[1], semo[1]).wait()
    for c in range(ct):
      pltpu.make_async_copy(
          table_hbm.at[idx_v.at[((SEQ - 1) >> 3) & 1, c, (SEQ - 1) & 7]],
          rows_v.at[0, pl.ds(c * 128, 128)],
          semg[0]).wait()

  return body


def kernel(inputs, token_table, pos_table):
  batch, seq = inputs.shape
  emb = token_table.shape[1]
  idx4 = (inputs.astype(jnp.int32).T
          .reshape(seq // 8, 8, batch // 128, 128)
          .transpose(0, 2, 1, 3))
  out5 = _make(batch)(idx4, token_table, pos_table.reshape(-1))
  return out5.transpose(2, 4, 0, 1, 3).reshape(batch, seq, emb)
